# baseline (speedup 1.0000x reference)
import functools

import jax
import jax.numpy as jnp
from jax import lax
from jax.experimental import pallas as pl
from jax.experimental.pallas import tpu as pltpu
from jax.experimental.pallas import tpu_sc as plsc

B, S, DIM, T, TOPK = 64, 64, 512, 1024, 4
_F32 = jnp.float32
NEG_INF = float("-inf")


def _dot(a, b):
    return lax.dot_general(a, b, (((1,), (0,)), ((), ())),
                           preferred_element_type=_F32)


def _dotT(a, b):
    return lax.dot_general(a, b, (((1,), (1,)), ((), ())),
                           preferred_element_type=_F32)


def _means_body(mi_ref, oi_ref):
    oi_ref[...] = jnp.mean(mi_ref[...], axis=1)


def _means(memory):
    TB = 64
    return pl.pallas_call(
        _means_body,
        grid=(T // TB,),
        in_specs=[pl.BlockSpec((TB, S, DIM), lambda i: (i, 0, 0))],
        out_specs=pl.BlockSpec((TB, DIM), lambda i: (i, 0)),
        out_shape=jax.ShapeDtypeStruct((T, DIM), _F32),
    )(memory)


def _route_body(rem_ref, m_ref, wr_ref, br_ref, wt_ref, bt_ref,
                idx_ref, rs_ref):
    q = jnp.mean(rem_ref[...], axis=1)
    qn = q / jnp.maximum(jnp.sqrt(jnp.sum(q * q, axis=1, keepdims=True)),
                         1e-8)
    m = m_ref[...]
    mn = m / jnp.maximum(jnp.sqrt(jnp.sum(m * m, axis=1, keepdims=True)),
                         1e-8)
    valid = (jnp.sum(m, axis=1, keepdims=True) != 0).astype(_F32)
    sim = _dotT(qn, mn * valid)

    avg_rem = _dotT(q, wr_ref[...]) + br_ref[...]
    u = _dot(avg_rem, wt_ref[...])
    c = jnp.sum(avg_rem * bt_ref[...], axis=1, keepdims=True)
    score_all = _dotT(u, m) + c

    iota_t = lax.broadcasted_iota(jnp.int32, (B, T), 1)
    lane = lax.broadcasted_iota(jnp.int32, (B, 128), 1)
    lanes = lax.broadcasted_iota(jnp.int32, (B, TOPK * _NSUB), 1)
    kk = lanes // _NSUB
    cc = lanes % _NSUB
    work = sim
    isub = jnp.zeros((B, TOPK * _NSUB), jnp.int32)
    scw = jnp.full((B, 128), NEG_INF, _F32)
    for j in range(TOPK):
        mx = jnp.max(work, axis=1, keepdims=True)
        amx = jnp.min(jnp.where(work == mx, iota_t, T), axis=1,
                      keepdims=True)
        sel = iota_t == amx
        s_j = jnp.sum(jnp.where(sel, score_all, 0.0), axis=1,
                      keepdims=True)
        isub = jnp.where(kk == j, amx * _NSUB, isub)
        scw = jnp.where(lane == j, s_j, scw)
        work = jnp.where(sel, NEG_INF, work)
    isub = isub + cc
    smx = jnp.max(scw, axis=1, keepdims=True)
    e = jnp.exp(scw - smx)
    rs = e / jnp.sum(e, axis=1, keepdims=True)
    idx_ref[...] = isub
    rs_ref[...] = rs


def _route(rem, m, Wr, br, Wt, bt):
    return pl.pallas_call(
        _route_body,
        out_shape=[
            jax.ShapeDtypeStruct((B, TOPK * _NSUB), jnp.int32),
            jax.ShapeDtypeStruct((B, 128), _F32),
        ],
    )(rem, m, Wr, br.reshape(1, DIM), Wt, bt.reshape(1, DIM))


_ROWS = B * TOPK
_RW = S * DIM
_NW = 32
_RPW = _ROWS // _NW


_NSUB = 64
_SUBW = _RW // _NSUB
_NSLOT = 3
_LA = 2


def _sc_gather_body(mem1_hbm, idx1_hbm, mem2_hbm, idx2_hbm,
                    out1_hbm, out2_hbm, idx_v, buf,
                    gs0, gs1, gs2, ss0, ss1, ss2):
    wid = lax.axis_index("s") * 2 + lax.axis_index("c")
    base = wid * _RPW
    nsub = _RPW * _NSUB
    pltpu.sync_copy(idx1_hbm.at[pl.ds(base * _NSUB, nsub)],
                    idx_v.at[pl.ds(0, nsub)])
    pltpu.sync_copy(idx2_hbm.at[pl.ds(base * _NSUB, nsub)],
                    idx_v.at[pl.ds(nsub, nsub)])
    gsem = (gs0, gs1, gs2)
    ssem = (ss0, ss1, ss2)
    nrows = 2 * _RPW
    gat = [None] * _NSLOT
    scat = [None] * _NSLOT
    for t in range(nrows + _LA):
        if t < nrows:
            s = t % _NSLOT
            if scat[s] is not None:
                scat[s].wait()
            mem = mem1_hbm if t < _RPW else mem2_hbm
            gat[s] = pltpu.async_copy(
                mem.at[idx_v.at[pl.ds(t * _NSUB, _NSUB)]],
                buf.at[s], gsem[s])
        if t >= _LA:
            j = t - _LA
            s2 = j % _NSLOT
            gat[s2].wait()
            out = out1_hbm if j < _RPW else out2_hbm
            jj = j % _RPW
            scat[s2] = pltpu.async_copy(
                buf.at[s2], out.at[pl.ds((base + jj) * _NSUB, _NSUB)],
                ssem[s2])
    for s in range(_NSLOT):
        if scat[s] is not None:
            scat[s].wait()


def _sc_gather1_body(mem_hbm, idx_hbm, out_hbm, idx_v, buf,
                     gs0, gs1, gs2, ss0, ss1, ss2):
    wid = lax.axis_index("s") * 2 + lax.axis_index("c")
    base = wid * _RPW
    nsub = _RPW * _NSUB
    pltpu.sync_copy(idx_hbm.at[pl.ds(base * _NSUB, nsub)], idx_v)
    gsem = (gs0, gs1, gs2)
    ssem = (ss0, ss1, ss2)
    gat = [None] * _NSLOT
    scat = [None] * _NSLOT
    for t in range(_RPW + _LA):
        if t < _RPW:
            s = t % _NSLOT
            if scat[s] is not None:
                scat[s].wait()
            gat[s] = pltpu.async_copy(
                mem_hbm.at[idx_v.at[pl.ds(t * _NSUB, _NSUB)]],
                buf.at[s], gsem[s])
        if t >= _LA:
            j = t - _LA
            s2 = j % _NSLOT
            gat[s2].wait()
            scat[s2] = pltpu.async_copy(
                buf.at[s2], out_hbm.at[pl.ds((base + j) * _NSUB, _NSUB)],
                ssem[s2])
    for s in range(_NSLOT):
        if scat[s] is not None:
            scat[s].wait()


def _sc_gather1(mem, idx):
    mesh = plsc.VectorSubcoreMesh(core_axis_name="c", subcore_axis_name="s")
    run = functools.partial(
        pl.kernel,
        out_type=jax.ShapeDtypeStruct((_ROWS * _NSUB, _SUBW), _F32),
        mesh=mesh,
        scratch_types=[
            pltpu.VMEM((_RPW * _NSUB,), jnp.int32),
            pltpu.VMEM((_NSLOT, _NSUB, _SUBW), _F32),
            pltpu.SemaphoreType.DMA,
            pltpu.SemaphoreType.DMA,
            pltpu.SemaphoreType.DMA,
            pltpu.SemaphoreType.DMA,
            pltpu.SemaphoreType.DMA,
            pltpu.SemaphoreType.DMA,
        ],
    )(_sc_gather1_body)
    return run(mem, idx)


def _sc_gather2(mem1, idx1, mem2, idx2):
    mesh = plsc.VectorSubcoreMesh(core_axis_name="c", subcore_axis_name="s")
    run = functools.partial(
        pl.kernel,
        out_type=(jax.ShapeDtypeStruct((_ROWS * _NSUB, _SUBW), _F32),
                  jax.ShapeDtypeStruct((_ROWS * _NSUB, _SUBW), _F32)),
        mesh=mesh,
        scratch_types=[
            pltpu.VMEM((2 * _RPW * _NSUB,), jnp.int32),
            pltpu.VMEM((_NSLOT, _NSUB, _SUBW), _F32),
            pltpu.SemaphoreType.DMA,
            pltpu.SemaphoreType.DMA,
            pltpu.SemaphoreType.DMA,
            pltpu.SemaphoreType.DMA,
            pltpu.SemaphoreType.DMA,
            pltpu.SemaphoreType.DMA,
        ],
    )(_sc_gather_body)
    return run(mem1, idx1, mem2, idx2)


_BBLK = 4


def _experts_body(g_ref, w1_ref, b1_ref, w2_ref, b2_ref, rs_ref,
                  quer_ref, flags_ref, comp_ref, full_ref):
    acc = jnp.zeros((_BBLK, S, DIM), _F32)
    for k in range(TOPK):
        rows = g_ref[:, k].reshape(_BBLK * S, DIM).astype(jnp.bfloat16)
        h = jnp.maximum(_dot(rows, w1_ref[k]) + b1_ref[k], 0.0)
        eo = _dot(h.astype(jnp.bfloat16), w2_ref[k]) + b2_ref[k]
        acc = acc + eo.reshape(_BBLK, S, DIM) * rs_ref[:, :, k:k + 1]
    miss = flags_ref[:, :, 0:1] > 0.5
    exist = flags_ref[:, :, 1:2] > 0.5
    comp_ref[...] = jnp.where(miss, acc, quer_ref[...])
    full_ref[...] = jnp.where(exist, acc, 0.0)


def _experts(g, W1t, b1, W2t, b2, rs3, quer, flags3):
    return pl.pallas_call(
        _experts_body,
        grid=(B // _BBLK,),
        in_specs=[
            pl.BlockSpec((_BBLK, TOPK, S, DIM), lambda i: (i, 0, 0, 0)),
            pl.BlockSpec((TOPK, DIM, DIM), lambda i: (0, 0, 0)),
            pl.BlockSpec((TOPK, 1, DIM), lambda i: (0, 0, 0)),
            pl.BlockSpec((TOPK, DIM, DIM), lambda i: (0, 0, 0)),
            pl.BlockSpec((TOPK, 1, DIM), lambda i: (0, 0, 0)),
            pl.BlockSpec((_BBLK, 1, 128), lambda i: (i, 0, 0)),
            pl.BlockSpec((_BBLK, S, DIM), lambda i: (i, 0, 0)),
            pl.BlockSpec((_BBLK, 1, 128), lambda i: (i, 0, 0)),
        ],
        out_specs=[
            pl.BlockSpec((_BBLK, S, DIM), lambda i: (i, 0, 0)),
            pl.BlockSpec((_BBLK, S, DIM), lambda i: (i, 0, 0)),
        ],
        out_shape=[
            jax.ShapeDtypeStruct((B, S, DIM), _F32),
            jax.ShapeDtypeStruct((B, S, DIM), _F32),
        ],
    )(g, W1t, b1, W2t, b2, rs3, quer, flags3)


def _expert_call(g, rsw, quer, flags, W1, b1, W2, b2):
    return _experts(g.reshape(B, TOPK, S, DIM),
                    W1.transpose(0, 2, 1).astype(jnp.bfloat16),
                    b1.reshape(TOPK, 1, DIM),
                    W2.transpose(0, 2, 1).astype(jnp.bfloat16),
                    b2.reshape(TOPK, 1, DIM),
                    rsw.reshape(B, 1, 128), quer, flags)


def kernel(image, text, m1, m2, memory_image, memory_text,
           ig_Wr, ig_br, ig_Wt, ig_bt, ig_W1, ig_b1, ig_W2, ig_b2,
           tg_Wr, tg_br, tg_Wt, tg_bt, tg_W1, tg_b1, tg_W2, tg_b2):
    text_exist = (m2 == 1)[:, 0]
    image_exist = (m1 == 1)[:, 0]
    img_missing = ((m1 == 0) & (m2 == 1))[:, 0]
    txt_missing = ((m2 == 0) & (m1 == 1))[:, 0]

    def mkflags(miss, exist):
        f = jnp.zeros((B, 128), _F32)
        f = f.at[:, 0].set(miss.astype(_F32))
        f = f.at[:, 1].set(exist.astype(_F32))
        return f.reshape(B, 1, 128)

    flags_img = mkflags(img_missing, text_exist)
    flags_txt = mkflags(txt_missing, image_exist)

    m_img = _means(memory_image)
    i1, r1 = _route(text, m_img, ig_Wr, ig_br, ig_Wt, ig_bt)
    g1 = _sc_gather1(memory_image.reshape(T * _NSUB, _SUBW),
                     i1.reshape(_ROWS * _NSUB))
    m_txt = _means(memory_text)
    i2, r2 = _route(image, m_txt, tg_Wr, tg_br, tg_Wt, tg_bt)
    g2 = _sc_gather1(memory_text.reshape(T * _NSUB, _SUBW),
                     i2.reshape(_ROWS * _NSUB))
    completed_image, gen_image_full = _expert_call(
        g1, r1, image, flags_img, ig_W1, ig_b1, ig_W2, ig_b2)
    completed_text, gen_text_full = _expert_call(
        g2, r2, text, flags_txt, tg_W1, tg_b1, tg_W2, tg_b2)

    return completed_image, completed_text, gen_image_full, gen_text_full


# speedup vs baseline: 1.0569x; 1.0569x over previous
import functools

import jax
import jax.numpy as jnp
from jax import lax
from jax.experimental import pallas as pl
from jax.experimental.pallas import tpu as pltpu
from jax.experimental.pallas import tpu_sc as plsc

B, S, DIM, T, TOPK = 64, 64, 512, 1024, 4
_F32 = jnp.float32
NEG_INF = float("-inf")


def _dot(a, b):
    return lax.dot_general(a, b, (((1,), (0,)), ((), ())),
                           preferred_element_type=_F32)


def _dotT(a, b):
    return lax.dot_general(a, b, (((1,), (1,)), ((), ())),
                           preferred_element_type=_F32)


def _means_body(mi_ref, mt_ref, oi_ref, ot_ref):
    oi_ref[...] = jnp.mean(mi_ref[...], axis=1)
    ot_ref[...] = jnp.mean(mt_ref[...], axis=1)


def _means(memory_image, memory_text):
    TB = 64
    return pl.pallas_call(
        _means_body,
        grid=(T // TB,),
        in_specs=[
            pl.BlockSpec((TB, S, DIM), lambda i: (i, 0, 0)),
            pl.BlockSpec((TB, S, DIM), lambda i: (i, 0, 0)),
        ],
        out_specs=[
            pl.BlockSpec((TB, DIM), lambda i: (i, 0)),
            pl.BlockSpec((TB, DIM), lambda i: (i, 0)),
        ],
        out_shape=[
            jax.ShapeDtypeStruct((T, DIM), _F32),
            jax.ShapeDtypeStruct((T, DIM), _F32),
        ],
    )(memory_image, memory_text)


def _route_body(rem_ref, m_ref, wr_ref, br_ref, wt_ref, bt_ref,
                idx_ref, rs_ref):
    q = jnp.mean(rem_ref[...], axis=1)
    qn = q / jnp.maximum(jnp.sqrt(jnp.sum(q * q, axis=1, keepdims=True)),
                         1e-8)
    m = m_ref[...]
    mn = m / jnp.maximum(jnp.sqrt(jnp.sum(m * m, axis=1, keepdims=True)),
                         1e-8)
    valid = (jnp.sum(m, axis=1, keepdims=True) != 0).astype(_F32)
    sim = _dotT(qn, mn * valid)

    avg_rem = _dotT(q, wr_ref[...]) + br_ref[...]
    u = _dot(avg_rem, wt_ref[...])
    c = jnp.sum(avg_rem * bt_ref[...], axis=1, keepdims=True)
    score_all = _dotT(u, m) + c

    iota_t = lax.broadcasted_iota(jnp.int32, (B, T), 1)
    lane = lax.broadcasted_iota(jnp.int32, (B, 128), 1)
    lanes = lax.broadcasted_iota(jnp.int32, (B, TOPK * _NSUB), 1)
    kk = lanes // _NSUB
    cc = lanes % _NSUB
    work = sim
    isub = jnp.zeros((B, TOPK * _NSUB), jnp.int32)
    scw = jnp.full((B, 128), NEG_INF, _F32)
    for j in range(TOPK):
        mx = jnp.max(work, axis=1, keepdims=True)
        amx = jnp.min(jnp.where(work == mx, iota_t, T), axis=1,
                      keepdims=True)
        sel = iota_t == amx
        s_j = jnp.sum(jnp.where(sel, score_all, 0.0), axis=1,
                      keepdims=True)
        isub = jnp.where(kk == j, amx * _NSUB, isub)
        scw = jnp.where(lane == j, s_j, scw)
        work = jnp.where(sel, NEG_INF, work)
    isub = isub + cc
    smx = jnp.max(scw, axis=1, keepdims=True)
    e = jnp.exp(scw - smx)
    rs = e / jnp.sum(e, axis=1, keepdims=True)
    idx_ref[...] = isub
    rs_ref[...] = rs


def _route(rem, m, Wr, br, Wt, bt):
    return pl.pallas_call(
        _route_body,
        out_shape=[
            jax.ShapeDtypeStruct((B, TOPK * _NSUB), jnp.int32),
            jax.ShapeDtypeStruct((B, 128), _F32),
        ],
    )(rem, m, Wr, br.reshape(1, DIM), Wt, bt.reshape(1, DIM))


_ROWS = B * TOPK
_RW = S * DIM
_NW = 32
_RPW = _ROWS // _NW


_NSUB = 64
_SUBW = _RW // _NSUB
_NSLOT = 3
_LA = 2


def _sc_gather_body(mem1_hbm, idx1_hbm, mem2_hbm, idx2_hbm,
                    out1_hbm, out2_hbm, idx_v, buf,
                    gs0, gs1, gs2, ss0, ss1, ss2):
    wid = lax.axis_index("s") * 2 + lax.axis_index("c")
    base = wid * _RPW
    nsub = _RPW * _NSUB
    pltpu.sync_copy(idx1_hbm.at[pl.ds(base * _NSUB, nsub)],
                    idx_v.at[pl.ds(0, nsub)])
    pltpu.sync_copy(idx2_hbm.at[pl.ds(base * _NSUB, nsub)],
                    idx_v.at[pl.ds(nsub, nsub)])
    gsem = (gs0, gs1, gs2)
    ssem = (ss0, ss1, ss2)
    nrows = 2 * _RPW
    gat = [None] * _NSLOT
    scat = [None] * _NSLOT
    for t in range(nrows + _LA):
        if t < nrows:
            s = t % _NSLOT
            if scat[s] is not None:
                scat[s].wait()
            mem = mem1_hbm if t < _RPW else mem2_hbm
            gat[s] = pltpu.async_copy(
                mem.at[idx_v.at[pl.ds(t * _NSUB, _NSUB)]],
                buf.at[s], gsem[s])
        if t >= _LA:
            j = t - _LA
            s2 = j % _NSLOT
            gat[s2].wait()
            out = out1_hbm if j < _RPW else out2_hbm
            jj = j % _RPW
            scat[s2] = pltpu.async_copy(
                buf.at[s2], out.at[pl.ds((base + jj) * _NSUB, _NSUB)],
                ssem[s2])
    for s in range(_NSLOT):
        if scat[s] is not None:
            scat[s].wait()


def _sc_gather1_body(mem_hbm, idx_hbm, out_hbm, idx_v, buf,
                     gs0, gs1, gs2, ss0, ss1, ss2):
    wid = lax.axis_index("s") * 2 + lax.axis_index("c")
    base = wid * _RPW
    nsub = _RPW * _NSUB
    pltpu.sync_copy(idx_hbm.at[pl.ds(base * _NSUB, nsub)], idx_v)
    gsem = (gs0, gs1, gs2)
    ssem = (ss0, ss1, ss2)
    gat = [None] * _NSLOT
    scat = [None] * _NSLOT
    for t in range(_RPW + _LA):
        if t < _RPW:
            s = t % _NSLOT
            if scat[s] is not None:
                scat[s].wait()
            gat[s] = pltpu.async_copy(
                mem_hbm.at[idx_v.at[pl.ds(t * _NSUB, _NSUB)]],
                buf.at[s], gsem[s])
        if t >= _LA:
            j = t - _LA
            s2 = j % _NSLOT
            gat[s2].wait()
            scat[s2] = pltpu.async_copy(
                buf.at[s2], out_hbm.at[pl.ds((base + j) * _NSUB, _NSUB)],
                ssem[s2])
    for s in range(_NSLOT):
        if scat[s] is not None:
            scat[s].wait()


def _sc_gather1(mem, idx):
    mesh = plsc.VectorSubcoreMesh(core_axis_name="c", subcore_axis_name="s")
    run = functools.partial(
        pl.kernel,
        out_type=jax.ShapeDtypeStruct((_ROWS * _NSUB, _SUBW), _F32),
        mesh=mesh,
        scratch_types=[
            pltpu.VMEM((_RPW * _NSUB,), jnp.int32),
            pltpu.VMEM((_NSLOT, _NSUB, _SUBW), _F32),
            pltpu.SemaphoreType.DMA,
            pltpu.SemaphoreType.DMA,
            pltpu.SemaphoreType.DMA,
            pltpu.SemaphoreType.DMA,
            pltpu.SemaphoreType.DMA,
            pltpu.SemaphoreType.DMA,
        ],
    )(_sc_gather1_body)
    return run(mem, idx)


def _sc_gather2(mem1, idx1, mem2, idx2):
    mesh = plsc.VectorSubcoreMesh(core_axis_name="c", subcore_axis_name="s")
    run = functools.partial(
        pl.kernel,
        out_type=(jax.ShapeDtypeStruct((_ROWS * _NSUB, _SUBW), _F32),
                  jax.ShapeDtypeStruct((_ROWS * _NSUB, _SUBW), _F32)),
        mesh=mesh,
        scratch_types=[
            pltpu.VMEM((2 * _RPW * _NSUB,), jnp.int32),
            pltpu.VMEM((_NSLOT, _NSUB, _SUBW), _F32),
            pltpu.SemaphoreType.DMA,
            pltpu.SemaphoreType.DMA,
            pltpu.SemaphoreType.DMA,
            pltpu.SemaphoreType.DMA,
            pltpu.SemaphoreType.DMA,
            pltpu.SemaphoreType.DMA,
        ],
    )(_sc_gather_body)
    return run(mem1, idx1, mem2, idx2)


_BBLK = 4


def _experts_body(g_ref, w1_ref, b1_ref, w2_ref, b2_ref, rs_ref,
                  quer_ref, flags_ref, comp_ref, full_ref):
    acc = jnp.zeros((_BBLK, S, DIM), _F32)
    for k in range(TOPK):
        rows = g_ref[:, k].reshape(_BBLK * S, DIM).astype(jnp.bfloat16)
        h = jnp.maximum(_dot(rows, w1_ref[k]) + b1_ref[k], 0.0)
        eo = _dot(h.astype(jnp.bfloat16), w2_ref[k]) + b2_ref[k]
        acc = acc + eo.reshape(_BBLK, S, DIM) * rs_ref[:, :, k:k + 1]
    miss = flags_ref[:, :, 0:1] > 0.5
    exist = flags_ref[:, :, 1:2] > 0.5
    comp_ref[...] = jnp.where(miss, acc, quer_ref[...])
    full_ref[...] = jnp.where(exist, acc, 0.0)


def _experts(g, W1t, b1, W2t, b2, rs3, quer, flags3):
    return pl.pallas_call(
        _experts_body,
        grid=(B // _BBLK,),
        in_specs=[
            pl.BlockSpec((_BBLK, TOPK, S, DIM), lambda i: (i, 0, 0, 0)),
            pl.BlockSpec((TOPK, DIM, DIM), lambda i: (0, 0, 0)),
            pl.BlockSpec((TOPK, 1, DIM), lambda i: (0, 0, 0)),
            pl.BlockSpec((TOPK, DIM, DIM), lambda i: (0, 0, 0)),
            pl.BlockSpec((TOPK, 1, DIM), lambda i: (0, 0, 0)),
            pl.BlockSpec((_BBLK, 1, 128), lambda i: (i, 0, 0)),
            pl.BlockSpec((_BBLK, S, DIM), lambda i: (i, 0, 0)),
            pl.BlockSpec((_BBLK, 1, 128), lambda i: (i, 0, 0)),
        ],
        out_specs=[
            pl.BlockSpec((_BBLK, S, DIM), lambda i: (i, 0, 0)),
            pl.BlockSpec((_BBLK, S, DIM), lambda i: (i, 0, 0)),
        ],
        out_shape=[
            jax.ShapeDtypeStruct((B, S, DIM), _F32),
            jax.ShapeDtypeStruct((B, S, DIM), _F32),
        ],
    )(g, W1t, b1, W2t, b2, rs3, quer, flags3)


def _expert_call(g, rsw, quer, flags, W1, b1, W2, b2):
    return _experts(g.reshape(B, TOPK, S, DIM),
                    W1.transpose(0, 2, 1).astype(jnp.bfloat16),
                    b1.reshape(TOPK, 1, DIM),
                    W2.transpose(0, 2, 1).astype(jnp.bfloat16),
                    b2.reshape(TOPK, 1, DIM),
                    rsw.reshape(B, 1, 128), quer, flags)


def kernel(image, text, m1, m2, memory_image, memory_text,
           ig_Wr, ig_br, ig_Wt, ig_bt, ig_W1, ig_b1, ig_W2, ig_b2,
           tg_Wr, tg_br, tg_Wt, tg_bt, tg_W1, tg_b1, tg_W2, tg_b2):
    text_exist = (m2 == 1)[:, 0]
    image_exist = (m1 == 1)[:, 0]
    img_missing = ((m1 == 0) & (m2 == 1))[:, 0]
    txt_missing = ((m2 == 0) & (m1 == 1))[:, 0]

    def mkflags(miss, exist):
        f = jnp.zeros((B, 128), _F32)
        f = f.at[:, 0].set(miss.astype(_F32))
        f = f.at[:, 1].set(exist.astype(_F32))
        return f.reshape(B, 1, 128)

    flags_img = mkflags(img_missing, text_exist)
    flags_txt = mkflags(txt_missing, image_exist)

    m_img, m_txt = _means(memory_image, memory_text)
    i1, r1 = _route(text, m_img, ig_Wr, ig_br, ig_Wt, ig_bt)
    i2, r2 = _route(image, m_txt, tg_Wr, tg_br, tg_Wt, tg_bt)
    g1 = _sc_gather1(memory_image.reshape(T * _NSUB, _SUBW),
                     i1.reshape(_ROWS * _NSUB))
    g2 = _sc_gather1(memory_text.reshape(T * _NSUB, _SUBW),
                     i2.reshape(_ROWS * _NSUB))
    completed_image, gen_image_full = _expert_call(
        g1, r1, image, flags_img, ig_W1, ig_b1, ig_W2, ig_b2)
    completed_text, gen_text_full = _expert_call(
        g2, r2, text, flags_txt, tg_W1, tg_b1, tg_W2, tg_b2)

    return completed_image, completed_text, gen_image_full, gen_text_full


# speedup vs baseline: 1.1587x; 1.0963x over previous
import functools

import jax
import jax.numpy as jnp
from jax import lax
from jax.experimental import pallas as pl
from jax.experimental.pallas import tpu as pltpu
from jax.experimental.pallas import tpu_sc as plsc

B, S, DIM, T, TOPK = 64, 64, 512, 1024, 4
_F32 = jnp.float32
NEG_INF = float("-inf")


def _dot(a, b):
    return lax.dot_general(a, b, (((1,), (0,)), ((), ())),
                           preferred_element_type=_F32)


def _dotT(a, b):
    return lax.dot_general(a, b, (((1,), (1,)), ((), ())),
                           preferred_element_type=_F32)


def _means_body(mi_ref, mt_ref, oi_ref, ot_ref):
    oi_ref[...] = jnp.mean(mi_ref[...], axis=1)
    ot_ref[...] = jnp.mean(mt_ref[...], axis=1)


def _means(memory_image, memory_text):
    TB = 64
    return pl.pallas_call(
        _means_body,
        grid=(T // TB,),
        in_specs=[
            pl.BlockSpec((TB, S, DIM), lambda i: (i, 0, 0)),
            pl.BlockSpec((TB, S, DIM), lambda i: (i, 0, 0)),
        ],
        out_specs=[
            pl.BlockSpec((TB, DIM), lambda i: (i, 0)),
            pl.BlockSpec((TB, DIM), lambda i: (i, 0)),
        ],
        out_shape=[
            jax.ShapeDtypeStruct((T, DIM), _F32),
            jax.ShapeDtypeStruct((T, DIM), _F32),
        ],
    )(memory_image, memory_text)


def _route_one(rem_ref, m_ref, wr_ref, br_ref, wt_ref, bt_ref,
               idx_ref, rs_ref):
    q = jnp.mean(rem_ref[...], axis=1)
    qn = q / jnp.maximum(jnp.sqrt(jnp.sum(q * q, axis=1, keepdims=True)),
                         1e-8)
    m = m_ref[...]
    mn = m / jnp.maximum(jnp.sqrt(jnp.sum(m * m, axis=1, keepdims=True)),
                         1e-8)
    valid = (jnp.sum(m, axis=1, keepdims=True) != 0).astype(_F32)
    sim = _dotT(qn, mn * valid)

    avg_rem = _dotT(q, wr_ref[...]) + br_ref[...]
    u = _dot(avg_rem, wt_ref[...])
    c = jnp.sum(avg_rem * bt_ref[...], axis=1, keepdims=True)
    score_all = _dotT(u, m) + c

    iota_t = lax.broadcasted_iota(jnp.int32, (B, T), 1)
    lane = lax.broadcasted_iota(jnp.int32, (B, 128), 1)
    lanes = lax.broadcasted_iota(jnp.int32, (B, TOPK * _NSUB), 1)
    kk = lanes // _NSUB
    cc = lanes % _NSUB
    work = sim
    isub = jnp.zeros((B, TOPK * _NSUB), jnp.int32)
    scw = jnp.full((B, 128), NEG_INF, _F32)
    for j in range(TOPK):
        mx = jnp.max(work, axis=1, keepdims=True)
        amx = jnp.min(jnp.where(work == mx, iota_t, T), axis=1,
                      keepdims=True)
        sel = iota_t == amx
        s_j = jnp.sum(jnp.where(sel, score_all, 0.0), axis=1,
                      keepdims=True)
        isub = jnp.where(kk == j, amx * _NSUB, isub)
        scw = jnp.where(lane == j, s_j, scw)
        work = jnp.where(sel, NEG_INF, work)
    isub = isub + cc
    smx = jnp.max(scw, axis=1, keepdims=True)
    e = jnp.exp(scw - smx)
    rs = e / jnp.sum(e, axis=1, keepdims=True)
    idx_ref[...] = isub
    rs_ref[...] = rs.reshape(B, 1, 128)


def _route2_body(text_ref, image_ref, mi_ref, mt_ref,
                 iwr_ref, ibr_ref, iwt_ref, ibt_ref,
                 twr_ref, tbr_ref, twt_ref, tbt_ref,
                 idx1_ref, rs1_ref, idx2_ref, rs2_ref):
    _route_one(text_ref, mi_ref, iwr_ref, ibr_ref, iwt_ref, ibt_ref,
               idx1_ref, rs1_ref)
    _route_one(image_ref, mt_ref, twr_ref, tbr_ref, twt_ref, tbt_ref,
               idx2_ref, rs2_ref)


def _route2(text, image, m_img, m_txt,
            iWr, ibr, iWt, ibt, tWr, tbr, tWt, tbt):
    return pl.pallas_call(
        _route2_body,
        out_shape=[
            jax.ShapeDtypeStruct((B, TOPK * _NSUB), jnp.int32),
            jax.ShapeDtypeStruct((B, 1, 128), _F32),
            jax.ShapeDtypeStruct((B, TOPK * _NSUB), jnp.int32),
            jax.ShapeDtypeStruct((B, 1, 128), _F32),
        ],
    )(text, image, m_img, m_txt,
      iWr, ibr.reshape(1, DIM), iWt, ibt.reshape(1, DIM),
      tWr, tbr.reshape(1, DIM), tWt, tbt.reshape(1, DIM))


_ROWS = B * TOPK
_RW = S * DIM
_NW = 32
_RPW = _ROWS // _NW


_NSUB = 64
_SUBW = _RW // _NSUB
_NSLOT = 3
_LA = 2
_BPW = B // _NW


def _sc_gather1_body(mem_hbm, idx_hbm, out_hbm, idx_v, buf,
                     gs0, gs1, gs2, ss0, ss1, ss2):
    wid = lax.axis_index("s") * 2 + lax.axis_index("c")
    base = wid * _RPW
    pltpu.sync_copy(idx_hbm.at[pl.ds(wid * _BPW, _BPW)], idx_v)
    gsem = (gs0, gs1, gs2)
    ssem = (ss0, ss1, ss2)
    gat = [None] * _NSLOT
    scat = [None] * _NSLOT
    for t in range(_RPW + _LA):
        if t < _RPW:
            s = t % _NSLOT
            if scat[s] is not None:
                scat[s].wait()
            gat[s] = pltpu.async_copy(
                mem_hbm.at[idx_v.at[t // TOPK,
                                    pl.ds((t % TOPK) * _NSUB, _NSUB)]],
                buf.at[s], gsem[s])
        if t >= _LA:
            j = t - _LA
            s2 = j % _NSLOT
            gat[s2].wait()
            scat[s2] = pltpu.async_copy(
                buf.at[s2], out_hbm.at[pl.ds((base + j) * _NSUB, _NSUB)],
                ssem[s2])
    for s in range(_NSLOT):
        if scat[s] is not None:
            scat[s].wait()


def _sc_gather1(mem, idx):
    mesh = plsc.VectorSubcoreMesh(core_axis_name="c", subcore_axis_name="s")
    run = functools.partial(
        pl.kernel,
        out_type=jax.ShapeDtypeStruct((_ROWS * _NSUB, _SUBW), _F32),
        mesh=mesh,
        scratch_types=[
            pltpu.VMEM((_BPW, TOPK * _NSUB), jnp.int32),
            pltpu.VMEM((_NSLOT, _NSUB, _SUBW), _F32),
            pltpu.SemaphoreType.DMA,
            pltpu.SemaphoreType.DMA,
            pltpu.SemaphoreType.DMA,
            pltpu.SemaphoreType.DMA,
            pltpu.SemaphoreType.DMA,
            pltpu.SemaphoreType.DMA,
        ],
    )(_sc_gather1_body)
    return run(mem, idx)


_BBLK = 8


def _experts_body(g_ref, w1_ref, b1_ref, w2_ref, b2_ref, rs_ref,
                  quer_ref, flags_ref, comp_ref, full_ref):
    acc = jnp.zeros((_BBLK, S, DIM), _F32)
    for k in range(TOPK):
        rows = g_ref[:, k].reshape(_BBLK * S, DIM).astype(jnp.bfloat16)
        h = jnp.maximum(_dotT(rows, w1_ref[k]) + b1_ref[k:k + 1, :], 0.0)
        eo = _dotT(h.astype(jnp.bfloat16), w2_ref[k]) + b2_ref[k:k + 1, :]
        acc = acc + eo.reshape(_BBLK, S, DIM) * rs_ref[:, :, k:k + 1]
    miss = flags_ref[:, :, 0:1] > 0.5
    exist = flags_ref[:, :, 1:2] > 0.5
    comp_ref[...] = jnp.where(miss, acc, quer_ref[...])
    full_ref[...] = jnp.where(exist, acc, 0.0)


def _experts(g, W1, b1, W2, b2, rs3, quer, flags3):
    return pl.pallas_call(
        _experts_body,
        grid=(B // _BBLK,),
        in_specs=[
            pl.BlockSpec((_BBLK, TOPK, S, DIM), lambda i: (i, 0, 0, 0)),
            pl.BlockSpec((TOPK, DIM, DIM), lambda i: (0, 0, 0)),
            pl.BlockSpec((TOPK, DIM), lambda i: (0, 0)),
            pl.BlockSpec((TOPK, DIM, DIM), lambda i: (0, 0, 0)),
            pl.BlockSpec((TOPK, DIM), lambda i: (0, 0)),
            pl.BlockSpec((_BBLK, 1, 128), lambda i: (i, 0, 0)),
            pl.BlockSpec((_BBLK, S, DIM), lambda i: (i, 0, 0)),
            pl.BlockSpec((_BBLK, 1, 128), lambda i: (i, 0, 0)),
        ],
        out_specs=[
            pl.BlockSpec((_BBLK, S, DIM), lambda i: (i, 0, 0)),
            pl.BlockSpec((_BBLK, S, DIM), lambda i: (i, 0, 0)),
        ],
        out_shape=[
            jax.ShapeDtypeStruct((B, S, DIM), _F32),
            jax.ShapeDtypeStruct((B, S, DIM), _F32),
        ],
    )(g, W1, b1, W2, b2, rs3, quer, flags3)


def _expert_call(g, rs3, quer, flags, W1, b1, W2, b2):
    return _experts(g.reshape(B, TOPK, S, DIM),
                    W1.astype(jnp.bfloat16), b1,
                    W2.astype(jnp.bfloat16), b2,
                    rs3, quer, flags)


def kernel(image, text, m1, m2, memory_image, memory_text,
           ig_Wr, ig_br, ig_Wt, ig_bt, ig_W1, ig_b1, ig_W2, ig_b2,
           tg_Wr, tg_br, tg_Wt, tg_bt, tg_W1, tg_b1, tg_W2, tg_b2):
    text_exist = (m2 == 1)[:, 0]
    image_exist = (m1 == 1)[:, 0]
    img_missing = ((m1 == 0) & (m2 == 1))[:, 0]
    txt_missing = ((m2 == 0) & (m1 == 1))[:, 0]

    lane = jnp.arange(128)[None, :]

    def mkflags(miss, exist):
        f = (jnp.where(lane == 0, miss[:, None].astype(_F32), 0.0)
             + jnp.where(lane == 1, exist[:, None].astype(_F32), 0.0))
        return f.reshape(B, 1, 128)

    flags_img = mkflags(img_missing, text_exist)
    flags_txt = mkflags(txt_missing, image_exist)

    m_img, m_txt = _means(memory_image, memory_text)
    i1, r1, i2, r2 = _route2(text, image, m_img, m_txt,
                             ig_Wr, ig_br, ig_Wt, ig_bt,
                             tg_Wr, tg_br, tg_Wt, tg_bt)
    g1 = _sc_gather1(memory_image.reshape(T * _NSUB, _SUBW), i1)
    g2 = _sc_gather1(memory_text.reshape(T * _NSUB, _SUBW), i2)
    completed_image, gen_image_full = _expert_call(
        g1, r1, image, flags_img, ig_W1, ig_b1, ig_W2, ig_b2)
    completed_text, gen_text_full = _expert_call(
        g2, r2, text, flags_txt, tg_W1, tg_b1, tg_W2, tg_b2)

    return completed_image, completed_text, gen_image_full, gen_text_full


# speedup vs baseline: 1.1813x; 1.0195x over previous
import functools

import jax
import jax.numpy as jnp
from jax import lax
from jax.experimental import pallas as pl
from jax.experimental.pallas import tpu as pltpu
from jax.experimental.pallas import tpu_sc as plsc

B, S, DIM, T, TOPK = 64, 64, 512, 1024, 4
_F32 = jnp.float32
NEG_INF = float("-inf")


def _dot(a, b):
    return lax.dot_general(a, b, (((1,), (0,)), ((), ())),
                           preferred_element_type=_F32)


def _dotT(a, b):
    return lax.dot_general(a, b, (((1,), (1,)), ((), ())),
                           preferred_element_type=_F32)


def _means_body(mi_ref, mt_ref, oi_ref, ot_ref):
    oi_ref[...] = jnp.mean(mi_ref[...], axis=1)
    ot_ref[...] = jnp.mean(mt_ref[...], axis=1)


def _means(memory_image, memory_text):
    TB = 64
    return pl.pallas_call(
        _means_body,
        grid=(T // TB,),
        in_specs=[
            pl.BlockSpec((TB, S, DIM), lambda i: (i, 0, 0)),
            pl.BlockSpec((TB, S, DIM), lambda i: (i, 0, 0)),
        ],
        out_specs=[
            pl.BlockSpec((TB, DIM), lambda i: (i, 0)),
            pl.BlockSpec((TB, DIM), lambda i: (i, 0)),
        ],
        out_shape=[
            jax.ShapeDtypeStruct((T, DIM), _F32),
            jax.ShapeDtypeStruct((T, DIM), _F32),
        ],
    )(memory_image, memory_text)


def _route_one(rem_ref, m_ref, wr_ref, br_ref, wt_ref, bt_ref,
               idx_ref, rs_ref):
    q = jnp.mean(rem_ref[...], axis=1)
    qn = q / jnp.maximum(jnp.sqrt(jnp.sum(q * q, axis=1, keepdims=True)),
                         1e-8)
    m = m_ref[...]
    mn = m / jnp.maximum(jnp.sqrt(jnp.sum(m * m, axis=1, keepdims=True)),
                         1e-8)
    valid = (jnp.sum(m, axis=1, keepdims=True) != 0).astype(_F32)
    sim = _dotT(qn, mn * valid)

    avg_rem = _dotT(q, wr_ref[...]) + br_ref[...]
    u = _dot(avg_rem, wt_ref[...])
    c = jnp.sum(avg_rem * bt_ref[...], axis=1, keepdims=True)
    score_all = _dotT(u, m) + c

    iota_t = lax.broadcasted_iota(jnp.int32, (B, T), 1)
    lane = lax.broadcasted_iota(jnp.int32, (B, 128), 1)
    lanes = lax.broadcasted_iota(jnp.int32, (B, TOPK * _NSUB), 1)
    kk = lanes // _NSUB
    cc = lanes % _NSUB
    work = sim
    isub = jnp.zeros((B, TOPK * _NSUB), jnp.int32)
    scw = jnp.full((B, 128), NEG_INF, _F32)
    for j in range(TOPK):
        mx = jnp.max(work, axis=1, keepdims=True)
        amx = jnp.min(jnp.where(work == mx, iota_t, T), axis=1,
                      keepdims=True)
        sel = iota_t == amx
        s_j = jnp.sum(jnp.where(sel, score_all, 0.0), axis=1,
                      keepdims=True)
        isub = jnp.where(kk == j, amx * _NSUB, isub)
        scw = jnp.where(lane == j, s_j, scw)
        work = jnp.where(sel, NEG_INF, work)
    isub = isub + cc
    smx = jnp.max(scw, axis=1, keepdims=True)
    e = jnp.exp(scw - smx)
    rs = e / jnp.sum(e, axis=1, keepdims=True)
    idx_ref[...] = isub
    rs_ref[...] = rs.reshape(B, 1, 128)


def _route2_body(text_ref, image_ref, mi_ref, mt_ref,
                 iwr_ref, ibr_ref, iwt_ref, ibt_ref,
                 twr_ref, tbr_ref, twt_ref, tbt_ref,
                 idx1_ref, rs1_ref, idx2_ref, rs2_ref):
    _route_one(text_ref, mi_ref, iwr_ref, ibr_ref, iwt_ref, ibt_ref,
               idx1_ref, rs1_ref)
    _route_one(image_ref, mt_ref, twr_ref, tbr_ref, twt_ref, tbt_ref,
               idx2_ref, rs2_ref)


def _route2(text, image, m_img, m_txt,
            iWr, ibr, iWt, ibt, tWr, tbr, tWt, tbt):
    return pl.pallas_call(
        _route2_body,
        out_shape=[
            jax.ShapeDtypeStruct((B, TOPK * _NSUB), jnp.int32),
            jax.ShapeDtypeStruct((B, 1, 128), _F32),
            jax.ShapeDtypeStruct((B, TOPK * _NSUB), jnp.int32),
            jax.ShapeDtypeStruct((B, 1, 128), _F32),
        ],
    )(text, image, m_img, m_txt,
      iWr, ibr.reshape(1, DIM), iWt, ibt.reshape(1, DIM),
      tWr, tbr.reshape(1, DIM), tWt, tbt.reshape(1, DIM))


_ROWS = B * TOPK
_RW = S * DIM
_NW = 32
_RPW = _ROWS // _NW


_NSUB = 64
_SUBW = _RW // _NSUB
_NSLOT = 3
_LA = 2
_BPW = B // _NW


def _sc_gather1_body(mem_hbm, idx_hbm, out_hbm, idx_v, buf,
                     gs0, gs1, gs2, ss0, ss1, ss2):
    wid = lax.axis_index("s") * 2 + lax.axis_index("c")
    base = wid * _RPW
    pltpu.sync_copy(idx_hbm.at[pl.ds(wid * _BPW, _BPW)], idx_v)
    gsem = (gs0, gs1, gs2)
    ssem = (ss0, ss1, ss2)
    gat = [None] * _NSLOT
    scat = [None] * _NSLOT
    for t in range(_RPW + _LA):
        if t < _RPW:
            s = t % _NSLOT
            if scat[s] is not None:
                scat[s].wait()
            gat[s] = pltpu.async_copy(
                mem_hbm.at[idx_v.at[t // TOPK,
                                    pl.ds((t % TOPK) * _NSUB, _NSUB)]],
                buf.at[s], gsem[s])
        if t >= _LA:
            j = t - _LA
            s2 = j % _NSLOT
            gat[s2].wait()
            scat[s2] = pltpu.async_copy(
                buf.at[s2], out_hbm.at[pl.ds((base + j) * _NSUB, _NSUB)],
                ssem[s2])
    for s in range(_NSLOT):
        if scat[s] is not None:
            scat[s].wait()


def _sc_gather1(mem, idx):
    mesh = plsc.VectorSubcoreMesh(core_axis_name="c", subcore_axis_name="s")
    run = functools.partial(
        pl.kernel,
        out_type=jax.ShapeDtypeStruct((_ROWS * _NSUB, _SUBW), _F32),
        mesh=mesh,
        scratch_types=[
            pltpu.VMEM((_BPW, TOPK * _NSUB), jnp.int32),
            pltpu.VMEM((_NSLOT, _NSUB, _SUBW), _F32),
            pltpu.SemaphoreType.DMA,
            pltpu.SemaphoreType.DMA,
            pltpu.SemaphoreType.DMA,
            pltpu.SemaphoreType.DMA,
            pltpu.SemaphoreType.DMA,
            pltpu.SemaphoreType.DMA,
        ],
    )(_sc_gather1_body)
    return run(mem, idx)


_BBLK = 8


def _experts_body(g_ref, w1_ref, b1_ref, w2_ref, b2_ref, rs_ref,
                  quer_ref, flags_ref, comp_ref, full_ref,
                  w1s_ref, w2s_ref):
    @pl.when(pl.program_id(0) == 0)
    def _():
        w1s_ref[...] = w1_ref[...].astype(jnp.bfloat16)
        w2s_ref[...] = w2_ref[...].astype(jnp.bfloat16)

    acc = jnp.zeros((_BBLK, S, DIM), _F32)
    for k in range(TOPK):
        rows = g_ref[:, k].reshape(_BBLK * S, DIM).astype(jnp.bfloat16)
        h = jnp.maximum(_dotT(rows, w1s_ref[k]) + b1_ref[k:k + 1, :], 0.0)
        eo = _dotT(h.astype(jnp.bfloat16), w2s_ref[k]) + b2_ref[k:k + 1, :]
        acc = acc + eo.reshape(_BBLK, S, DIM) * rs_ref[:, :, k:k + 1]
    miss = flags_ref[:, :, 0:1] > 0.5
    exist = flags_ref[:, :, 1:2] > 0.5
    comp_ref[...] = jnp.where(miss, acc, quer_ref[...])
    full_ref[...] = jnp.where(exist, acc, 0.0)


def _experts(g, W1, b1, W2, b2, rs3, quer, flags3):
    return pl.pallas_call(
        _experts_body,
        grid=(B // _BBLK,),
        in_specs=[
            pl.BlockSpec((_BBLK, TOPK, S, DIM), lambda i: (i, 0, 0, 0)),
            pl.BlockSpec((TOPK, DIM, DIM), lambda i: (0, 0, 0)),
            pl.BlockSpec((TOPK, DIM), lambda i: (0, 0)),
            pl.BlockSpec((TOPK, DIM, DIM), lambda i: (0, 0, 0)),
            pl.BlockSpec((TOPK, DIM), lambda i: (0, 0)),
            pl.BlockSpec((_BBLK, 1, 128), lambda i: (i, 0, 0)),
            pl.BlockSpec((_BBLK, S, DIM), lambda i: (i, 0, 0)),
            pl.BlockSpec((_BBLK, 1, 128), lambda i: (i, 0, 0)),
        ],
        out_specs=[
            pl.BlockSpec((_BBLK, S, DIM), lambda i: (i, 0, 0)),
            pl.BlockSpec((_BBLK, S, DIM), lambda i: (i, 0, 0)),
        ],
        out_shape=[
            jax.ShapeDtypeStruct((B, S, DIM), _F32),
            jax.ShapeDtypeStruct((B, S, DIM), _F32),
        ],
        scratch_shapes=[
            pltpu.VMEM((TOPK, DIM, DIM), jnp.bfloat16),
            pltpu.VMEM((TOPK, DIM, DIM), jnp.bfloat16),
        ],
    )(g, W1, b1, W2, b2, rs3, quer, flags3)


def _expert_call(g, rs3, quer, flags, W1, b1, W2, b2):
    return _experts(g.reshape(B, TOPK, S, DIM), W1, b1, W2, b2,
                    rs3, quer, flags)


def kernel(image, text, m1, m2, memory_image, memory_text,
           ig_Wr, ig_br, ig_Wt, ig_bt, ig_W1, ig_b1, ig_W2, ig_b2,
           tg_Wr, tg_br, tg_Wt, tg_bt, tg_W1, tg_b1, tg_W2, tg_b2):
    text_exist = (m2 == 1)[:, 0]
    image_exist = (m1 == 1)[:, 0]
    img_missing = ((m1 == 0) & (m2 == 1))[:, 0]
    txt_missing = ((m2 == 0) & (m1 == 1))[:, 0]

    lane = jnp.arange(128)[None, :]

    def mkflags(miss, exist):
        f = (jnp.where(lane == 0, miss[:, None].astype(_F32), 0.0)
             + jnp.where(lane == 1, exist[:, None].astype(_F32), 0.0))
        return f.reshape(B, 1, 128)

    flags_img = mkflags(img_missing, text_exist)
    flags_txt = mkflags(txt_missing, image_exist)

    m_img, m_txt = _means(memory_image, memory_text)
    i1, r1, i2, r2 = _route2(text, image, m_img, m_txt,
                             ig_Wr, ig_br, ig_Wt, ig_bt,
                             tg_Wr, tg_br, tg_Wt, tg_bt)
    g1 = _sc_gather1(memory_image.reshape(T * _NSUB, _SUBW), i1)
    g2 = _sc_gather1(memory_text.reshape(T * _NSUB, _SUBW), i2)
    completed_image, gen_image_full = _expert_call(
        g1, r1, image, flags_img, ig_W1, ig_b1, ig_W2, ig_b2)
    completed_text, gen_text_full = _expert_call(
        g2, r2, text, flags_txt, tg_W1, tg_b1, tg_W2, tg_b2)

    return completed_image, completed_text, gen_image_full, gen_text_full


# speedup vs baseline: 1.2092x; 1.0236x over previous
import functools

import jax
import jax.numpy as jnp
from jax import lax
from jax.experimental import pallas as pl
from jax.experimental.pallas import tpu as pltpu
from jax.experimental.pallas import tpu_sc as plsc

B, S, DIM, T, TOPK = 64, 64, 512, 1024, 4
_F32 = jnp.float32
NEG_INF = float("-inf")


def _dot(a, b):
    return lax.dot_general(a, b, (((1,), (0,)), ((), ())),
                           preferred_element_type=_F32)


def _dotT(a, b):
    return lax.dot_general(a, b, (((1,), (1,)), ((), ())),
                           preferred_element_type=_F32)


def _route_one(rem_ref, m, wr_ref, br_ref, wt_ref, bt_ref,
               idx_ref, rs_ref):
    q = jnp.mean(rem_ref[...], axis=1)
    qn = q / jnp.maximum(jnp.sqrt(jnp.sum(q * q, axis=1, keepdims=True)),
                         1e-8)
    mn = m / jnp.maximum(jnp.sqrt(jnp.sum(m * m, axis=1, keepdims=True)),
                         1e-8)
    valid = (jnp.sum(m, axis=1, keepdims=True) != 0).astype(_F32)
    sim = _dotT(qn, mn * valid)

    avg_rem = _dotT(q, wr_ref[...]) + br_ref[...]
    u = _dot(avg_rem, wt_ref[...])
    c = jnp.sum(avg_rem * bt_ref[...], axis=1, keepdims=True)
    score_all = _dotT(u, m) + c

    iota_t = lax.broadcasted_iota(jnp.int32, (B, T), 1)
    lane = lax.broadcasted_iota(jnp.int32, (B, 128), 1)
    lanes = lax.broadcasted_iota(jnp.int32, (B, TOPK * _NSUB), 1)
    kk = lanes // _NSUB
    cc = lanes % _NSUB
    work = sim
    isub = jnp.zeros((B, TOPK * _NSUB), jnp.int32)
    scw = jnp.full((B, 128), NEG_INF, _F32)
    for j in range(TOPK):
        mx = jnp.max(work, axis=1, keepdims=True)
        amx = jnp.min(jnp.where(work == mx, iota_t, T), axis=1,
                      keepdims=True)
        sel = iota_t == amx
        s_j = jnp.sum(jnp.where(sel, score_all, 0.0), axis=1,
                      keepdims=True)
        isub = jnp.where(kk == j, amx * _NSUB, isub)
        scw = jnp.where(lane == j, s_j, scw)
        work = jnp.where(sel, NEG_INF, work)
    isub = isub + cc
    smx = jnp.max(scw, axis=1, keepdims=True)
    e = jnp.exp(scw - smx)
    rs = e / jnp.sum(e, axis=1, keepdims=True)
    idx_ref[...] = isub
    rs_ref[...] = rs.reshape(B, 1, 128)


_MTB = 32


def _means_route_body(mi_ref, mt_ref, text_ref, image_ref,
                      iwr_ref, ibr_ref, iwt_ref, ibt_ref,
                      twr_ref, tbr_ref, twt_ref, tbt_ref,
                      idx1_ref, rs1_ref, idx2_ref, rs2_ref,
                      ms_i, ms_t):
    i = pl.program_id(0)
    ms_i[pl.ds(i * _MTB, _MTB), :] = jnp.mean(mi_ref[...], axis=1)
    ms_t[pl.ds(i * _MTB, _MTB), :] = jnp.mean(mt_ref[...], axis=1)

    @pl.when(i == T // _MTB - 1)
    def _():
        _route_one(text_ref, ms_i[...], iwr_ref, ibr_ref, iwt_ref, ibt_ref,
                   idx1_ref, rs1_ref)
        _route_one(image_ref, ms_t[...], twr_ref, tbr_ref, twt_ref, tbt_ref,
                   idx2_ref, rs2_ref)


def _means_route(memory_image, memory_text, text, image,
                 iWr, ibr, iWt, ibt, tWr, tbr, tWt, tbt):
    const3 = lambda i: (0, 0, 0)
    const2 = lambda i: (0, 0)
    return pl.pallas_call(
        _means_route_body,
        grid=(T // _MTB,),
        in_specs=[
            pl.BlockSpec((_MTB, S, DIM), lambda i: (i, 0, 0)),
            pl.BlockSpec((_MTB, S, DIM), lambda i: (i, 0, 0)),
            pl.BlockSpec((B, S, DIM), const3),
            pl.BlockSpec((B, S, DIM), const3),
            pl.BlockSpec((DIM, DIM), const2),
            pl.BlockSpec((1, DIM), const2),
            pl.BlockSpec((DIM, DIM), const2),
            pl.BlockSpec((1, DIM), const2),
            pl.BlockSpec((DIM, DIM), const2),
            pl.BlockSpec((1, DIM), const2),
            pl.BlockSpec((DIM, DIM), const2),
            pl.BlockSpec((1, DIM), const2),
        ],
        out_specs=[
            pl.BlockSpec((B, TOPK * _NSUB), const2),
            pl.BlockSpec((B, 1, 128), const3),
            pl.BlockSpec((B, TOPK * _NSUB), const2),
            pl.BlockSpec((B, 1, 128), const3),
        ],
        out_shape=[
            jax.ShapeDtypeStruct((B, TOPK * _NSUB), jnp.int32),
            jax.ShapeDtypeStruct((B, 1, 128), _F32),
            jax.ShapeDtypeStruct((B, TOPK * _NSUB), jnp.int32),
            jax.ShapeDtypeStruct((B, 1, 128), _F32),
        ],
        scratch_shapes=[
            pltpu.VMEM((T, DIM), _F32),
            pltpu.VMEM((T, DIM), _F32),
        ],
    )(memory_image, memory_text, text, image,
      iWr, ibr.reshape(1, DIM), iWt, ibt.reshape(1, DIM),
      tWr, tbr.reshape(1, DIM), tWt, tbt.reshape(1, DIM))


_ROWS = B * TOPK
_RW = S * DIM
_NW = 32
_RPW = _ROWS // _NW


_NSUB = 64
_SUBW = _RW // _NSUB
_NSLOT = 3
_LA = 2
_BPW = B // _NW


def _sc_gather1_body(mem_hbm, idx_hbm, out_hbm, idx_v, buf,
                     gs0, gs1, gs2, ss0, ss1, ss2):
    wid = lax.axis_index("s") * 2 + lax.axis_index("c")
    base = wid * _RPW
    pltpu.sync_copy(idx_hbm.at[pl.ds(wid * _BPW, _BPW)], idx_v)
    gsem = (gs0, gs1, gs2)
    ssem = (ss0, ss1, ss2)
    gat = [None] * _NSLOT
    scat = [None] * _NSLOT
    for t in range(_RPW + _LA):
        if t < _RPW:
            s = t % _NSLOT
            if scat[s] is not None:
                scat[s].wait()
            gat[s] = pltpu.async_copy(
                mem_hbm.at[idx_v.at[t // TOPK,
                                    pl.ds((t % TOPK) * _NSUB, _NSUB)]],
                buf.at[s], gsem[s])
        if t >= _LA:
            j = t - _LA
            s2 = j % _NSLOT
            gat[s2].wait()
            scat[s2] = pltpu.async_copy(
                buf.at[s2], out_hbm.at[pl.ds((base + j) * _NSUB, _NSUB)],
                ssem[s2])
    for s in range(_NSLOT):
        if scat[s] is not None:
            scat[s].wait()


def _sc_gather1(mem, idx):
    mesh = plsc.VectorSubcoreMesh(core_axis_name="c", subcore_axis_name="s")
    run = functools.partial(
        pl.kernel,
        out_type=jax.ShapeDtypeStruct((_ROWS * _NSUB, _SUBW), _F32),
        mesh=mesh,
        scratch_types=[
            pltpu.VMEM((_BPW, TOPK * _NSUB), jnp.int32),
            pltpu.VMEM((_NSLOT, _NSUB, _SUBW), _F32),
            pltpu.SemaphoreType.DMA,
            pltpu.SemaphoreType.DMA,
            pltpu.SemaphoreType.DMA,
            pltpu.SemaphoreType.DMA,
            pltpu.SemaphoreType.DMA,
            pltpu.SemaphoreType.DMA,
        ],
    )(_sc_gather1_body)
    return run(mem, idx)


_BBLK = 8


def _experts_body(g_ref, w1_ref, b1_ref, w2_ref, b2_ref, rs_ref,
                  quer_ref, flags_ref, comp_ref, full_ref,
                  w1s_ref, w2s_ref):
    @pl.when(pl.program_id(0) == 0)
    def _():
        w1s_ref[...] = w1_ref[...].astype(jnp.bfloat16)
        w2s_ref[...] = w2_ref[...].astype(jnp.bfloat16)

    acc = jnp.zeros((_BBLK, S, DIM), _F32)
    for k in range(TOPK):
        rows = g_ref[:, k].reshape(_BBLK * S, DIM).astype(jnp.bfloat16)
        h = jnp.maximum(_dotT(rows, w1s_ref[k]) + b1_ref[k:k + 1, :], 0.0)
        eo = _dotT(h.astype(jnp.bfloat16), w2s_ref[k]) + b2_ref[k:k + 1, :]
        acc = acc + eo.reshape(_BBLK, S, DIM) * rs_ref[:, :, k:k + 1]
    miss = flags_ref[:, :, 0:1] > 0.5
    exist = flags_ref[:, :, 1:2] > 0.5
    comp_ref[...] = jnp.where(miss, acc, quer_ref[...])
    full_ref[...] = jnp.where(exist, acc, 0.0)


def _experts(g, W1, b1, W2, b2, rs3, quer, flags3):
    return pl.pallas_call(
        _experts_body,
        grid=(B // _BBLK,),
        in_specs=[
            pl.BlockSpec((_BBLK, TOPK, S, DIM), lambda i: (i, 0, 0, 0)),
            pl.BlockSpec((TOPK, DIM, DIM), lambda i: (0, 0, 0)),
            pl.BlockSpec((TOPK, DIM), lambda i: (0, 0)),
            pl.BlockSpec((TOPK, DIM, DIM), lambda i: (0, 0, 0)),
            pl.BlockSpec((TOPK, DIM), lambda i: (0, 0)),
            pl.BlockSpec((_BBLK, 1, 128), lambda i: (i, 0, 0)),
            pl.BlockSpec((_BBLK, S, DIM), lambda i: (i, 0, 0)),
            pl.BlockSpec((_BBLK, 1, 128), lambda i: (i, 0, 0)),
        ],
        out_specs=[
            pl.BlockSpec((_BBLK, S, DIM), lambda i: (i, 0, 0)),
            pl.BlockSpec((_BBLK, S, DIM), lambda i: (i, 0, 0)),
        ],
        out_shape=[
            jax.ShapeDtypeStruct((B, S, DIM), _F32),
            jax.ShapeDtypeStruct((B, S, DIM), _F32),
        ],
        scratch_shapes=[
            pltpu.VMEM((TOPK, DIM, DIM), jnp.bfloat16),
            pltpu.VMEM((TOPK, DIM, DIM), jnp.bfloat16),
        ],
    )(g, W1, b1, W2, b2, rs3, quer, flags3)


def _expert_call(g, rs3, quer, flags, W1, b1, W2, b2):
    return _experts(g.reshape(B, TOPK, S, DIM), W1, b1, W2, b2,
                    rs3, quer, flags)


def kernel(image, text, m1, m2, memory_image, memory_text,
           ig_Wr, ig_br, ig_Wt, ig_bt, ig_W1, ig_b1, ig_W2, ig_b2,
           tg_Wr, tg_br, tg_Wt, tg_bt, tg_W1, tg_b1, tg_W2, tg_b2):
    text_exist = (m2 == 1)[:, 0]
    image_exist = (m1 == 1)[:, 0]
    img_missing = ((m1 == 0) & (m2 == 1))[:, 0]
    txt_missing = ((m2 == 0) & (m1 == 1))[:, 0]

    lane = jnp.arange(128)[None, :]

    def mkflags(miss, exist):
        f = (jnp.where(lane == 0, miss[:, None].astype(_F32), 0.0)
             + jnp.where(lane == 1, exist[:, None].astype(_F32), 0.0))
        return f.reshape(B, 1, 128)

    flags_img = mkflags(img_missing, text_exist)
    flags_txt = mkflags(txt_missing, image_exist)

    i1, r1, i2, r2 = _means_route(memory_image, memory_text, text, image,
                                  ig_Wr, ig_br, ig_Wt, ig_bt,
                                  tg_Wr, tg_br, tg_Wt, tg_bt)
    g1 = _sc_gather1(memory_image.reshape(T * _NSUB, _SUBW), i1)
    g2 = _sc_gather1(memory_text.reshape(T * _NSUB, _SUBW), i2)
    completed_image, gen_image_full = _expert_call(
        g1, r1, image, flags_img, ig_W1, ig_b1, ig_W2, ig_b2)
    completed_text, gen_text_full = _expert_call(
        g2, r2, text, flags_txt, tg_W1, tg_b1, tg_W2, tg_b2)

    return completed_image, completed_text, gen_image_full, gen_text_full


# speedup vs baseline: 1.2730x; 1.0528x over previous
import functools

import jax
import jax.numpy as jnp
from jax import lax
from jax.experimental import pallas as pl
from jax.experimental.pallas import tpu as pltpu
from jax.experimental.pallas import tpu_sc as plsc

B, S, DIM, T, TOPK = 64, 64, 512, 1024, 4
_F32 = jnp.float32
NEG_INF = float("-inf")


def _dot(a, b):
    return lax.dot_general(a, b, (((1,), (0,)), ((), ())),
                           preferred_element_type=_F32)


def _dotT(a, b):
    return lax.dot_general(a, b, (((1,), (1,)), ((), ())),
                           preferred_element_type=_F32)


def _route_one(rem_ref, m, wr_ref, br_ref, wt_ref, bt_ref,
               idx_ref, rs_ref):
    q = jnp.mean(rem_ref[...], axis=1)
    qn = q / jnp.maximum(jnp.sqrt(jnp.sum(q * q, axis=1, keepdims=True)),
                         1e-8)
    mn = m / jnp.maximum(jnp.sqrt(jnp.sum(m * m, axis=1, keepdims=True)),
                         1e-8)
    valid = (jnp.sum(m, axis=1, keepdims=True) != 0).astype(_F32)
    sim = _dotT(qn, mn * valid)

    avg_rem = _dotT(q, wr_ref[...]) + br_ref[...]
    u = _dot(avg_rem, wt_ref[...])
    c = jnp.sum(avg_rem * bt_ref[...], axis=1, keepdims=True)
    score_all = _dotT(u, m) + c

    iota_t = lax.broadcasted_iota(jnp.int32, (B, T), 1)
    lane = lax.broadcasted_iota(jnp.int32, (B, 128), 1)
    lanes = lax.broadcasted_iota(jnp.int32, (B, TOPK * _NSUB), 1)
    kk = lanes // _NSUB
    cc = lanes % _NSUB
    work = sim
    isub = jnp.zeros((B, TOPK * _NSUB), jnp.int32)
    scw = jnp.full((B, 128), NEG_INF, _F32)
    for j in range(TOPK):
        mx = jnp.max(work, axis=1, keepdims=True)
        amx = jnp.min(jnp.where(work == mx, iota_t, T), axis=1,
                      keepdims=True)
        sel = iota_t == amx
        s_j = jnp.sum(jnp.where(sel, score_all, 0.0), axis=1,
                      keepdims=True)
        isub = jnp.where(kk == j, amx * _NSUB, isub)
        scw = jnp.where(lane == j, s_j, scw)
        work = jnp.where(sel, NEG_INF, work)
    isub = isub + cc
    smx = jnp.max(scw, axis=1, keepdims=True)
    e = jnp.exp(scw - smx)
    rs = e / jnp.sum(e, axis=1, keepdims=True)
    idx_ref[...] = isub
    rs_ref[...] = rs.reshape(B, 1, 128)


_MTB = 32


def _means_route_body(mi_ref, mt_ref, text_ref, image_ref,
                      iwr_ref, ibr_ref, iwt_ref, ibt_ref,
                      twr_ref, tbr_ref, twt_ref, tbt_ref,
                      idx1_ref, rs1_ref, idx2_ref, rs2_ref,
                      ms_i, ms_t):
    i = pl.program_id(0)
    ms_i[pl.ds(i * _MTB, _MTB), :] = jnp.mean(mi_ref[...], axis=1)
    ms_t[pl.ds(i * _MTB, _MTB), :] = jnp.mean(mt_ref[...], axis=1)

    @pl.when(i == T // _MTB - 1)
    def _():
        _route_one(text_ref, ms_i[...], iwr_ref, ibr_ref, iwt_ref, ibt_ref,
                   idx1_ref, rs1_ref)
        _route_one(image_ref, ms_t[...], twr_ref, tbr_ref, twt_ref, tbt_ref,
                   idx2_ref, rs2_ref)


def _means_route(memory_image, memory_text, text, image,
                 iWr, ibr, iWt, ibt, tWr, tbr, tWt, tbt):
    const3 = lambda i: (0, 0, 0)
    const2 = lambda i: (0, 0)
    return pl.pallas_call(
        _means_route_body,
        grid=(T // _MTB,),
        in_specs=[
            pl.BlockSpec((_MTB, S, DIM), lambda i: (i, 0, 0)),
            pl.BlockSpec((_MTB, S, DIM), lambda i: (i, 0, 0)),
            pl.BlockSpec((B, S, DIM), const3),
            pl.BlockSpec((B, S, DIM), const3),
            pl.BlockSpec((DIM, DIM), const2),
            pl.BlockSpec((1, DIM), const2),
            pl.BlockSpec((DIM, DIM), const2),
            pl.BlockSpec((1, DIM), const2),
            pl.BlockSpec((DIM, DIM), const2),
            pl.BlockSpec((1, DIM), const2),
            pl.BlockSpec((DIM, DIM), const2),
            pl.BlockSpec((1, DIM), const2),
        ],
        out_specs=[
            pl.BlockSpec((B, TOPK * _NSUB), const2),
            pl.BlockSpec((B, 1, 128), const3),
            pl.BlockSpec((B, TOPK * _NSUB), const2),
            pl.BlockSpec((B, 1, 128), const3),
        ],
        out_shape=[
            jax.ShapeDtypeStruct((B, TOPK * _NSUB), jnp.int32),
            jax.ShapeDtypeStruct((B, 1, 128), _F32),
            jax.ShapeDtypeStruct((B, TOPK * _NSUB), jnp.int32),
            jax.ShapeDtypeStruct((B, 1, 128), _F32),
        ],
        scratch_shapes=[
            pltpu.VMEM((T, DIM), _F32),
            pltpu.VMEM((T, DIM), _F32),
        ],
    )(memory_image, memory_text, text, image,
      iWr, ibr.reshape(1, DIM), iWt, ibt.reshape(1, DIM),
      tWr, tbr.reshape(1, DIM), tWt, tbt.reshape(1, DIM))


_ROWS = B * TOPK
_RW = S * DIM
_NW = 32
_RPW = _ROWS // _NW


_NSUB = 64
_SUBW = _RW // _NSUB
_NSLOT = 3
_LA = 2
_BPW = B // _NW


def _sc_gather1_body(mem_hbm, idx_hbm, act_hbm, out_hbm, idx_v, act_v, buf,
                     gs0, gs1, gs2, ss0, ss1, ss2):
    wid = lax.axis_index("s") * 2 + lax.axis_index("c")
    base = wid * _RPW
    pltpu.sync_copy(idx_hbm.at[pl.ds(wid * _BPW, _BPW)], idx_v)
    pltpu.sync_copy(act_hbm.at[pl.ds(wid * _BPW, _BPW)], act_v)
    gsem = (gs0, gs1, gs2)
    ssem = (ss0, ss1, ss2)

    for bo in range(_BPW):
        actf = act_v[bo][0]

        @pl.when(actf > 0)
        def _(bo=bo):
            gat = [None] * _NSLOT
            scat = [None] * _NSLOT
            for t in range(TOPK + 1):
                if t < TOPK:
                    s = t % _NSLOT
                    if scat[s] is not None:
                        scat[s].wait()
                    gat[s] = pltpu.async_copy(
                        mem_hbm.at[idx_v.at[bo, pl.ds(t * _NSUB, _NSUB)]],
                        buf.at[s], gsem[s])
                if t >= 1:
                    j = t - 1
                    s2 = j % _NSLOT
                    gat[s2].wait()
                    scat[s2] = pltpu.async_copy(
                        buf.at[s2],
                        out_hbm.at[pl.ds((base + bo * TOPK + j) * _NSUB,
                                         _NSUB)],
                        ssem[s2])
            for s in range(_NSLOT):
                if scat[s] is not None:
                    scat[s].wait()


def _sc_gather1(mem, idx, act):
    mesh = plsc.VectorSubcoreMesh(core_axis_name="c", subcore_axis_name="s")
    run = functools.partial(
        pl.kernel,
        out_type=jax.ShapeDtypeStruct((_ROWS * _NSUB, _SUBW), _F32),
        mesh=mesh,
        scratch_types=[
            pltpu.VMEM((_BPW, TOPK * _NSUB), jnp.int32),
            pltpu.VMEM((_BPW, 16), jnp.int32),
            pltpu.VMEM((_NSLOT, _NSUB, _SUBW), _F32),
            pltpu.SemaphoreType.DMA,
            pltpu.SemaphoreType.DMA,
            pltpu.SemaphoreType.DMA,
            pltpu.SemaphoreType.DMA,
            pltpu.SemaphoreType.DMA,
            pltpu.SemaphoreType.DMA,
        ],
    )(_sc_gather1_body)
    return run(mem, idx, act)


_BBLK = 8


def _experts_body(g_ref, w1_ref, b1_ref, w2_ref, b2_ref, rs_ref,
                  quer_ref, flags_ref, comp_ref, full_ref,
                  w1s_ref, w2s_ref):
    @pl.when(pl.program_id(0) == 0)
    def _():
        w1s_ref[...] = w1_ref[...].astype(jnp.bfloat16)
        w2s_ref[...] = w2_ref[...].astype(jnp.bfloat16)

    acc = jnp.zeros((_BBLK, S, DIM), _F32)
    for k in range(TOPK):
        rows = g_ref[:, k].reshape(_BBLK * S, DIM).astype(jnp.bfloat16)
        h = jnp.maximum(_dotT(rows, w1s_ref[k]) + b1_ref[k:k + 1, :], 0.0)
        eo = _dotT(h.astype(jnp.bfloat16), w2s_ref[k]) + b2_ref[k:k + 1, :]
        acc = acc + eo.reshape(_BBLK, S, DIM) * rs_ref[:, :, k:k + 1]
    miss = flags_ref[:, :, 0:1] > 0.5
    exist = flags_ref[:, :, 1:2] > 0.5
    comp_ref[...] = jnp.where(miss, acc, quer_ref[...])
    full_ref[...] = jnp.where(exist, acc, 0.0)


def _experts(g, W1, b1, W2, b2, rs3, quer, flags3):
    return pl.pallas_call(
        _experts_body,
        grid=(B // _BBLK,),
        in_specs=[
            pl.BlockSpec((_BBLK, TOPK, S, DIM), lambda i: (i, 0, 0, 0)),
            pl.BlockSpec((TOPK, DIM, DIM), lambda i: (0, 0, 0)),
            pl.BlockSpec((TOPK, DIM), lambda i: (0, 0)),
            pl.BlockSpec((TOPK, DIM, DIM), lambda i: (0, 0, 0)),
            pl.BlockSpec((TOPK, DIM), lambda i: (0, 0)),
            pl.BlockSpec((_BBLK, 1, 128), lambda i: (i, 0, 0)),
            pl.BlockSpec((_BBLK, S, DIM), lambda i: (i, 0, 0)),
            pl.BlockSpec((_BBLK, 1, 128), lambda i: (i, 0, 0)),
        ],
        out_specs=[
            pl.BlockSpec((_BBLK, S, DIM), lambda i: (i, 0, 0)),
            pl.BlockSpec((_BBLK, S, DIM), lambda i: (i, 0, 0)),
        ],
        out_shape=[
            jax.ShapeDtypeStruct((B, S, DIM), _F32),
            jax.ShapeDtypeStruct((B, S, DIM), _F32),
        ],
        scratch_shapes=[
            pltpu.VMEM((TOPK, DIM, DIM), jnp.bfloat16),
            pltpu.VMEM((TOPK, DIM, DIM), jnp.bfloat16),
        ],
    )(g, W1, b1, W2, b2, rs3, quer, flags3)


def _expert_call(g, rs3, quer, flags, W1, b1, W2, b2):
    return _experts(g.reshape(B, TOPK, S, DIM), W1, b1, W2, b2,
                    rs3, quer, flags)


def kernel(image, text, m1, m2, memory_image, memory_text,
           ig_Wr, ig_br, ig_Wt, ig_bt, ig_W1, ig_b1, ig_W2, ig_b2,
           tg_Wr, tg_br, tg_Wt, tg_bt, tg_W1, tg_b1, tg_W2, tg_b2):
    text_exist = (m2 == 1)[:, 0]
    image_exist = (m1 == 1)[:, 0]
    img_missing = ((m1 == 0) & (m2 == 1))[:, 0]
    txt_missing = ((m2 == 0) & (m1 == 1))[:, 0]

    lane = jnp.arange(128)[None, :]

    def mkflags(miss, exist):
        f = (jnp.where(lane == 0, miss[:, None].astype(_F32), 0.0)
             + jnp.where(lane == 1, exist[:, None].astype(_F32), 0.0))
        return f.reshape(B, 1, 128)

    flags_img = mkflags(img_missing, text_exist)
    flags_txt = mkflags(txt_missing, image_exist)

    i1, r1, i2, r2 = _means_route(memory_image, memory_text, text, image,
                                  ig_Wr, ig_br, ig_Wt, ig_bt,
                                  tg_Wr, tg_br, tg_Wt, tg_bt)
    act1 = jnp.broadcast_to(text_exist.astype(jnp.int32)[:, None], (B, 16))
    act2 = jnp.broadcast_to(image_exist.astype(jnp.int32)[:, None], (B, 16))
    g1 = _sc_gather1(memory_image.reshape(T * _NSUB, _SUBW), i1, act1)
    g2 = _sc_gather1(memory_text.reshape(T * _NSUB, _SUBW), i2, act2)
    completed_image, gen_image_full = _expert_call(
        g1, r1, image, flags_img, ig_W1, ig_b1, ig_W2, ig_b2)
    completed_text, gen_text_full = _expert_call(
        g2, r2, text, flags_txt, tg_W1, tg_b1, tg_W2, tg_b2)

    return completed_image, completed_text, gen_image_full, gen_text_full
